# baseline (device time: 191093 ns/iter reference)
import jax
import jax.numpy as jnp
from jax import lax
from jax.experimental import pallas as pl
from jax.experimental.pallas import tpu as pltpu

N_DEV = 16
LANES = 4

_sem_signal = getattr(pl, "semaphore_signal", None) or pltpu.semaphore_signal
_sem_wait = getattr(pl, "semaphore_wait", None) or pltpu.semaphore_wait
_CompilerParams = getattr(pltpu, "CompilerParams", None) or pltpu.TPUCompilerParams


def kernel(x, w_mat):
    M, k_per = x.shape
    _, N = w_mat.shape
    Mc = M // N_DEV
    Nl = N // LANES

    def body(x_ref, w_ref, out_ref,
             sb0, rb0, sb1, rb1, sb2, rb2, sb3, rb3,
             amax_ref, amax_recv,
             ss0, rs0, ss1, rs1, ss2, rs2, ss3, rs3,
             asend_sems, arecv_sems,
             cred0, cred1, cred2, cred3):
        my = lax.axis_index("i")
        left = lax.rem(my - 1 + N_DEV, N_DEV)
        right = lax.rem(my + 1, N_DEV)

        lanes = [
            (sb0, rb0, ss0, rs0, cred0, right, left),
            (sb1, rb1, ss1, rs1, cred1, right, left),
            (sb2, rb2, ss2, rs2, cred2, left, right),
            (sb3, rb3, ss3, rs3, cred3, left, right),
        ]

        barrier_sem = pltpu.get_barrier_semaphore()
        for nbr in (left, right):
            _sem_signal(barrier_sem, inc=1, device_id=(nbr,),
                        device_id_type=pl.DeviceIdType.MESH)
        _sem_wait(barrier_sem, 2)

        def rdma(lane, slot, to_dst=True):
            sb, rb, ss, rs_, _, dst, src = lanes[lane]
            return pltpu.make_async_remote_copy(
                src_ref=sb.at[slot], dst_ref=rb.at[slot],
                send_sem=ss.at[slot], recv_sem=rs_.at[slot],
                device_id=(dst if to_dst else src,),
                device_id_type=pl.DeviceIdType.MESH)

        for s in range(N_DEV):
            cR = lax.rem(my - 1 - s + 2 * N_DEV, N_DEV)
            cL = lax.rem(my + 1 + s, N_DEV)
            xR = x_ref[pl.ds(cR * Mc, Mc), :]
            xL = x_ref[pl.ds(cL * Mc, Mc), :]
            for lane in range(LANES):
                sb, rb, ss, rs_, cred, dst, src = lanes[lane]
                xc = xR if lane < 2 else xL
                p = jnp.dot(xc, w_ref[:, lane * Nl:(lane + 1) * Nl],
                            preferred_element_type=jnp.float32)
                if s == 0:
                    acc = p
                else:
                    rslot = (s - 1) % 2
                    rdma(lane, rslot).wait_recv()
                    acc = p + rb[rslot, :, :]
                    if s <= N_DEV - 3:
                        _sem_signal(cred, inc=1, device_id=(src,),
                                    device_id_type=pl.DeviceIdType.MESH)
                if s < N_DEV - 1:
                    sslot = s % 2
                    if s >= 2:
                        rdma(lane, sslot).wait_send()
                        _sem_wait(cred, 1)
                    sb[sslot, :, :] = acc
                    rdma(lane, sslot).start()
                else:
                    out_ref[:, lane * Nl:(lane + 1) * Nl] = acc

        for lane in range(LANES):
            for sl in (1, 0):
                rdma(lane, sl).wait_send()

        local_amax = jnp.max(jnp.abs(out_ref[:, :]))
        amax_ref[:, :] = jnp.full((8, 128), local_amax, jnp.float32)
        for j in range(1, N_DEV):
            partner = my ^ j
            pltpu.make_async_remote_copy(
                src_ref=amax_ref, dst_ref=amax_recv.at[j],
                send_sem=asend_sems.at[j], recv_sem=arecv_sems.at[j],
                device_id=(partner,),
                device_id_type=pl.DeviceIdType.MESH).start()
        gmax = amax_ref[:, :]
        for j in range(1, N_DEV):
            partner = my ^ j
            pltpu.make_async_remote_copy(
                src_ref=amax_ref, dst_ref=amax_recv.at[j],
                send_sem=asend_sems.at[j], recv_sem=arecv_sems.at[j],
                device_id=(partner,),
                device_id_type=pl.DeviceIdType.MESH).wait_recv()
            gmax = jnp.maximum(gmax, amax_recv[j, :, :])
        for j in range(1, N_DEV):
            partner = my ^ j
            pltpu.make_async_remote_copy(
                src_ref=amax_ref, dst_ref=amax_recv.at[j],
                send_sem=asend_sems.at[j], recv_sem=arecv_sems.at[j],
                device_id=(partner,),
                device_id_type=pl.DeviceIdType.MESH).wait_send()

        scale = jnp.max(gmax) / 127.0
        q = jnp.clip(jnp.round(out_ref[:, :] / scale), -127.0, 127.0)
        out_ref[:, :] = q * scale

    return pl.pallas_call(
        body,
        out_shape=jax.ShapeDtypeStruct((Mc, N), jnp.float32),
        in_specs=[pl.BlockSpec(memory_space=pltpu.VMEM),
                  pl.BlockSpec(memory_space=pltpu.VMEM)],
        out_specs=pl.BlockSpec(memory_space=pltpu.VMEM),
        scratch_shapes=(
            [pltpu.VMEM((2, Mc, Nl), jnp.float32)] * 8 +
            [pltpu.VMEM((8, 128), jnp.float32),
             pltpu.VMEM((N_DEV, 8, 128), jnp.float32)] +
            [pltpu.SemaphoreType.DMA((2,))] * 8 +
            [pltpu.SemaphoreType.DMA((N_DEV,)),
             pltpu.SemaphoreType.DMA((N_DEV,))] +
            [pltpu.SemaphoreType.REGULAR] * 4
        ),
        compiler_params=_CompilerParams(collective_id=0),
    )(x, w_mat)


# device time: 188358 ns/iter; 1.0145x vs baseline; 1.0145x over previous
import jax
import jax.numpy as jnp
from jax import lax
from jax.experimental import pallas as pl
from jax.experimental.pallas import tpu as pltpu

N_DEV = 16
LANES = 4
SLOTS = 3

_sem_signal = getattr(pl, "semaphore_signal", None) or pltpu.semaphore_signal
_sem_wait = getattr(pl, "semaphore_wait", None) or pltpu.semaphore_wait
_CompilerParams = getattr(pltpu, "CompilerParams", None) or pltpu.TPUCompilerParams


def kernel(x, w_mat):
    M, k_per = x.shape
    _, N = w_mat.shape
    Mc = M // N_DEV
    Nl = N // LANES

    def body(x_ref, w_ref, out_ref,
             sb0, rb0, sb1, rb1, sb2, rb2, sb3, rb3,
             amax_ref, amax_recv,
             ss0, rs0, ss1, rs1, ss2, rs2, ss3, rs3,
             asend_sems, arecv_sems,
             cred0, cred1, cred2, cred3):
        my = lax.axis_index("i")
        left = lax.rem(my - 1 + N_DEV, N_DEV)
        right = lax.rem(my + 1, N_DEV)

        lanes = [
            (sb0, rb0, ss0, rs0, cred0, right, left, 0 * Nl, True),
            (sb1, rb1, ss1, rs1, cred1, left, right, 2 * Nl, False),
            (sb2, rb2, ss2, rs2, cred2, right, left, 1 * Nl, True),
            (sb3, rb3, ss3, rs3, cred3, left, right, 3 * Nl, False),
        ]

        barrier_sem = pltpu.get_barrier_semaphore()
        for nbr in (left, right):
            _sem_signal(barrier_sem, inc=1, device_id=(nbr,),
                        device_id_type=pl.DeviceIdType.MESH)
        _sem_wait(barrier_sem, 2)

        def rdma(lane, slot):
            sb, rb, ss, rs_, _, dst, src, col, goes_right = lanes[lane]
            return pltpu.make_async_remote_copy(
                src_ref=sb.at[slot], dst_ref=rb.at[slot],
                send_sem=ss.at[slot], recv_sem=rs_.at[slot],
                device_id=(dst,), device_id_type=pl.DeviceIdType.MESH)

        for s in range(N_DEV):
            cR = lax.rem(my - 1 - s + 2 * N_DEV, N_DEV)
            cL = lax.rem(my + 1 + s, N_DEV)
            xR = x_ref[pl.ds(cR * Mc, Mc), :]
            xL = x_ref[pl.ds(cL * Mc, Mc), :]
            for lane in range(LANES):
                sb, rb, ss, rs_, cred, dst, src, col, goes_right = lanes[lane]
                xc = xR if goes_right else xL
                p = jnp.dot(xc, w_ref[:, col:col + Nl],
                            preferred_element_type=jnp.float32)
                if s == 0:
                    acc = p
                else:
                    rslot = (s - 1) % SLOTS
                    rdma(lane, rslot).wait_recv()
                    acc = p + rb[rslot, :, :]
                    if s <= N_DEV - 1 - SLOTS:
                        _sem_signal(cred, inc=1, device_id=(src,),
                                    device_id_type=pl.DeviceIdType.MESH)
                if s < N_DEV - 1:
                    sslot = s % SLOTS
                    if s >= SLOTS:
                        rdma(lane, sslot).wait_send()
                        _sem_wait(cred, 1)
                    sb[sslot, :, :] = acc
                    rdma(lane, sslot).start()
                else:
                    out_ref[:, col:col + Nl] = acc

        for lane in range(LANES):
            for s in range(N_DEV - 1 - SLOTS, N_DEV - 1):
                rdma(lane, s % SLOTS).wait_send()

        local_amax = jnp.max(jnp.abs(out_ref[:, :]))
        amax_ref[:, :] = jnp.full((8, 128), local_amax, jnp.float32)
        for j in range(1, N_DEV):
            partner = my ^ j
            pltpu.make_async_remote_copy(
                src_ref=amax_ref, dst_ref=amax_recv.at[j],
                send_sem=asend_sems.at[j], recv_sem=arecv_sems.at[j],
                device_id=(partner,),
                device_id_type=pl.DeviceIdType.MESH).start()
        gmax = amax_ref[:, :]
        for j in range(1, N_DEV):
            partner = my ^ j
            pltpu.make_async_remote_copy(
                src_ref=amax_ref, dst_ref=amax_recv.at[j],
                send_sem=asend_sems.at[j], recv_sem=arecv_sems.at[j],
                device_id=(partner,),
                device_id_type=pl.DeviceIdType.MESH).wait_recv()
            gmax = jnp.maximum(gmax, amax_recv[j, :, :])
        for j in range(1, N_DEV):
            partner = my ^ j
            pltpu.make_async_remote_copy(
                src_ref=amax_ref, dst_ref=amax_recv.at[j],
                send_sem=asend_sems.at[j], recv_sem=arecv_sems.at[j],
                device_id=(partner,),
                device_id_type=pl.DeviceIdType.MESH).wait_send()

        scale = jnp.max(gmax) / 127.0
        q = jnp.clip(jnp.round(out_ref[:, :] / scale), -127.0, 127.0)
        out_ref[:, :] = q * scale

    return pl.pallas_call(
        body,
        out_shape=jax.ShapeDtypeStruct((Mc, N), jnp.float32),
        in_specs=[pl.BlockSpec(memory_space=pltpu.VMEM),
                  pl.BlockSpec(memory_space=pltpu.VMEM)],
        out_specs=pl.BlockSpec(memory_space=pltpu.VMEM),
        scratch_shapes=(
            [pltpu.VMEM((SLOTS, Mc, Nl), jnp.float32)] * 8 +
            [pltpu.VMEM((8, 128), jnp.float32),
             pltpu.VMEM((N_DEV, 8, 128), jnp.float32)] +
            [pltpu.SemaphoreType.DMA((SLOTS,))] * 8 +
            [pltpu.SemaphoreType.DMA((N_DEV,)),
             pltpu.SemaphoreType.DMA((N_DEV,))] +
            [pltpu.SemaphoreType.REGULAR] * 4
        ),
        compiler_params=_CompilerParams(collective_id=0),
    )(x, w_mat)
